# baseline (speedup 1.0000x reference)
import functools

import jax
import jax.numpy as jnp
from jax import lax
from jax.experimental import pallas as pl
from jax.experimental.pallas import tpu as pltpu
from jax.experimental.pallas import tpu_sc as plsc

DIM = 64
HIST = 50
NUM_CORES = 2
NUM_SUBCORES = 16
NW = NUM_CORES * NUM_SUBCORES
LANES = 16
BLK = 128


def _body(idx_hbm, ktab_hbm, pret_hbm, out_hbm, idx_v, ktab_v,
          ocol0, ocol1, scol0, scol1,
          rows0, rows1, pblk0, pblk1, gsem0, gsem1, wsem0, wsem1):
    bpw = BLK * HIST
    wid = lax.axis_index("s") * NUM_CORES + lax.axis_index("c")
    base = wid * bpw
    ocol = (ocol0, ocol1)
    scol = (scol0, scol1)
    rows = (rows0, rows1)
    pblk = (pblk0, pblk1)
    gsem = (gsem0, gsem1)
    wsem = (wsem0, wsem1)

    pltpu.sync_copy(idx_hbm.at[pl.ds(base, bpw)], idx_v)
    pltpu.sync_copy(ktab_hbm, ktab_v)

    lane = lax.iota(jnp.int32, LANES)

    def _mk_col(h, b):
        for s in range(BLK // LANES):
            p = (lane + s * LANES) * HIST + h
            v = plsc.load_gather(idx_v, [p])
            ocol[b][pl.ds(s * LANES, LANES)] = v
            scol[b][pl.ds(s * LANES, LANES)] = jnp.maximum(v - 2, 0)
        return pltpu.async_copy(pret_hbm.at[scol[b]], rows[b], gsem[b])

    def _fixup(b):
        for j in range(BLK // LANES):
            v = ocol[b][pl.ds(j * LANES, LANES)]

            @pl.when(jnp.min(v) < 2)
            def _():
                m = v < 2
                vs = jnp.minimum(v, 1)
                r16 = lane + j * LANES
                for d in range(DIM):
                    col = jnp.full((LANES,), d, jnp.int32)
                    x = plsc.load_gather(ktab_v, [vs, col])
                    plsc.store_scatter(rows[b], [r16, col], x, mask=m)

    def _transpose(b):
        for c in range(BLK):
            cc = jnp.full((LANES,), c, jnp.int32)
            for k in range(DIM // LANES):
                v = rows[b][c, pl.ds(k * LANES, LANES)]
                plsc.store_scatter(pblk[b], [lane + k * LANES, cc], v)

    def _write(h, b):
        for q in range(8):
            r0 = ((h * 8 + q) * NW + wid) * 8
            pltpu.async_copy(
                pblk[b].at[pl.ds(q * 8, 8)],
                out_hbm.at[pl.ds(r0, 8)],
                wsem[b],
            )

    def _wait_gather(b):
        pltpu.make_async_copy(pret_hbm.at[scol[b]], rows[b], gsem[b]).wait()

    def _drain_writes(b):
        for q in range(8):
            pltpu.make_async_copy(
                pblk[b].at[pl.ds(q * 8, 8)],
                out_hbm.at[pl.ds(q * 8, 8)],
                wsem[b],
            ).wait()

    _mk_col(0, 0)

    def _iter(i, carry):
        h0 = 2 * i
        _mk_col(h0 + 1, 1)
        _wait_gather(0)
        _fixup(0)
        pl.when(i > 0)(lambda: _drain_writes(0))
        _transpose(0)
        _write(h0, 0)
        def _prefetch():
            _mk_col(h0 + 2, 0)

        pl.when(i + 1 < HIST // 2)(_prefetch)
        _wait_gather(1)
        _fixup(1)
        pl.when(i > 0)(lambda: _drain_writes(1))
        _transpose(1)
        _write(h0 + 1, 1)
        return carry

    lax.fori_loop(0, HIST // 2, _iter, 0)
    _drain_writes(0)
    _drain_writes(1)


def _gather(idx, ktab, pret):
    n = idx.shape[0]
    mesh = plsc.VectorSubcoreMesh(core_axis_name="c", subcore_axis_name="s")
    nrows = n * DIM // 128
    return pl.kernel(
        _body,
        out_type=jax.ShapeDtypeStruct((nrows, 128), jnp.float32),
        mesh=mesh,
        compiler_params=pltpu.CompilerParams(
            needs_layout_passes=False, use_tc_tiling_on_sc=False
        ),
        scratch_types=[
            pltpu.VMEM((BLK * HIST,), jnp.int32),
            pltpu.VMEM((2, DIM), jnp.float32),
            pltpu.VMEM((BLK,), jnp.int32),
            pltpu.VMEM((BLK,), jnp.int32),
            pltpu.VMEM((BLK,), jnp.int32),
            pltpu.VMEM((BLK,), jnp.int32),
            pltpu.VMEM((BLK, DIM), jnp.float32),
            pltpu.VMEM((BLK, DIM), jnp.float32),
            pltpu.VMEM((DIM, BLK), jnp.float32),
            pltpu.VMEM((DIM, BLK), jnp.float32),
            pltpu.SemaphoreType.DMA,
            pltpu.SemaphoreType.DMA,
            pltpu.SemaphoreType.DMA,
            pltpu.SemaphoreType.DMA,
        ],
    )(idx, ktab, pret)


def kernel(inputs, kernel, pretrained):
    b, h = inputs.shape
    idx = inputs.reshape(-1).astype(jnp.int32)
    out2 = _gather(idx, kernel, pretrained)
    o5 = out2.reshape(h, DIM // 8, b // 128, 8, 128)
    return o5.transpose(2, 4, 0, 1, 3).reshape(b, h, DIM)


# speedup vs baseline: 1.4061x; 1.4061x over previous
import functools

import jax
import jax.numpy as jnp
from jax import lax
from jax.experimental import pallas as pl
from jax.experimental.pallas import tpu as pltpu
from jax.experimental.pallas import tpu_sc as plsc

DIM = 64
HIST = 50
NUM_CORES = 2
NUM_SUBCORES = 16
NW = NUM_CORES * NUM_SUBCORES
LANES = 16
BLK = 128


def _body(idx_hbm, ktab_hbm, pret_hbm, out_hbm, idx_v, ktab_v,
          ocol0, ocol1, scol0, scol1,
          rows0, rows1, pblk0, pblk1, gsem0, gsem1, wsem0, wsem1):
    bpw = BLK * HIST
    wid = lax.axis_index("s") * NUM_CORES + lax.axis_index("c")
    base = wid * bpw
    ocol = (ocol0, ocol1)
    scol = (scol0, scol1)
    rows = (rows0, rows1)
    pblk = (pblk0, pblk1)
    gsem = (gsem0, gsem1)
    wsem = (wsem0, wsem1)

    pltpu.sync_copy(idx_hbm.at[pl.ds(base, bpw)], idx_v)
    pltpu.sync_copy(ktab_hbm, ktab_v)

    lane = lax.iota(jnp.int32, LANES)

    def _mk_col(h, b):
        for s in range(BLK // LANES):
            p = (lane + s * LANES) * HIST + h
            v = plsc.load_gather(idx_v, [p])
            ocol[b][pl.ds(s * LANES, LANES)] = v
            scol[b][pl.ds(s * LANES, LANES)] = jnp.maximum(v - 2, 0)
        return pltpu.async_copy(pret_hbm.at[scol[b]], rows[b], gsem[b])

    def _fixup(b):
        for j in range(BLK // LANES):
            v = ocol[b][pl.ds(j * LANES, LANES)]

            @pl.when(jnp.min(v) < 2)
            def _():
                m = v < 2
                vs = jnp.minimum(v, 1)
                r16 = lane + j * LANES
                for d in range(DIM):
                    col = jnp.full((LANES,), d, jnp.int32)
                    x = plsc.load_gather(ktab_v, [vs, col])
                    plsc.store_scatter(rows[b], [r16, col], x, mask=m)

    def _transpose(b):
        for c in range(BLK):
            cc = jnp.full((LANES,), c, jnp.int32)
            for k in range(DIM // LANES):
                v = rows[b][c, pl.ds(k * LANES, LANES)]
                plsc.store_scatter(pblk[b], [lane + k * LANES, cc], v)

    def _write(h, b):
        for q in range(8):
            r0 = ((h * 8 + q) * NW + wid) * 8
            pltpu.async_copy(
                pblk[b].at[pl.ds(q * 8, 8), pl.ds(0, BLK)],
                out_hbm.at[pl.ds(r0, 8)],
                wsem[b],
            )

    def _wait_gather(b):
        pltpu.make_async_copy(pret_hbm.at[scol[b]], rows[b], gsem[b]).wait()

    def _drain_writes(b):
        for q in range(8):
            pltpu.make_async_copy(
                pblk[b].at[pl.ds(q * 8, 8), pl.ds(0, BLK)],
                out_hbm.at[pl.ds(q * 8, 8)],
                wsem[b],
            ).wait()

    _mk_col(0, 0)

    def _iter(i, carry):
        h0 = 2 * i
        _mk_col(h0 + 1, 1)
        _wait_gather(0)
        _fixup(0)
        pl.when(i > 0)(lambda: _drain_writes(0))
        _transpose(0)
        _write(h0, 0)
        def _prefetch():
            _mk_col(h0 + 2, 0)

        pl.when(i + 1 < HIST // 2)(_prefetch)
        _wait_gather(1)
        _fixup(1)
        pl.when(i > 0)(lambda: _drain_writes(1))
        _transpose(1)
        _write(h0 + 1, 1)
        return carry

    lax.fori_loop(0, HIST // 2, _iter, 0)
    _drain_writes(0)
    _drain_writes(1)


def _gather(idx, ktab, pret):
    n = idx.shape[0]
    mesh = plsc.VectorSubcoreMesh(core_axis_name="c", subcore_axis_name="s")
    nrows = n * DIM // 128
    return pl.kernel(
        _body,
        out_type=jax.ShapeDtypeStruct((nrows, 128), jnp.float32),
        mesh=mesh,
        compiler_params=pltpu.CompilerParams(
            needs_layout_passes=False, use_tc_tiling_on_sc=False
        ),
        scratch_types=[
            pltpu.VMEM((BLK * HIST,), jnp.int32),
            pltpu.VMEM((2, DIM), jnp.float32),
            pltpu.VMEM((BLK,), jnp.int32),
            pltpu.VMEM((BLK,), jnp.int32),
            pltpu.VMEM((BLK,), jnp.int32),
            pltpu.VMEM((BLK,), jnp.int32),
            pltpu.VMEM((BLK, DIM), jnp.float32),
            pltpu.VMEM((BLK, DIM), jnp.float32),
            pltpu.VMEM((DIM, BLK + 1), jnp.float32),
            pltpu.VMEM((DIM, BLK + 1), jnp.float32),
            pltpu.SemaphoreType.DMA,
            pltpu.SemaphoreType.DMA,
            pltpu.SemaphoreType.DMA,
            pltpu.SemaphoreType.DMA,
        ],
    )(idx, ktab, pret)


def kernel(inputs, kernel, pretrained):
    b, h = inputs.shape
    idx = inputs.reshape(-1).astype(jnp.int32)
    out2 = _gather(idx, kernel, pretrained)
    o5 = out2.reshape(h, DIM // 8, b // 128, 8, 128)
    return o5.transpose(2, 4, 0, 1, 3).reshape(b, h, DIM)


# speedup vs baseline: 2.3201x; 1.6501x over previous
import functools

import jax
import jax.numpy as jnp
from jax import lax
from jax.experimental import pallas as pl
from jax.experimental.pallas import tpu as pltpu
from jax.experimental.pallas import tpu_sc as plsc

DIM = 64
HIST = 50
NUM_CORES = 2
NUM_SUBCORES = 16
NW = NUM_CORES * NUM_SUBCORES
LANES = 16
BLK = 128


def _body(idx_hbm, ktab_hbm, pret_hbm, out_hbm, idx_v, ktab_v,
          ocol0, ocol1, scol0, scol1,
          rows0, rows1, pblk0, pblk1, gsem0, gsem1, wsem0, wsem1):
    bpw = BLK * HIST
    wid = lax.axis_index("s") * NUM_CORES + lax.axis_index("c")
    base = wid * bpw
    ocol = (ocol0, ocol1)
    scol = (scol0, scol1)
    rows = (rows0, rows1)
    pblk = (pblk0, pblk1)
    gsem = (gsem0, gsem1)
    wsem = (wsem0, wsem1)

    pltpu.sync_copy(idx_hbm.at[pl.ds(base, bpw)], idx_v)
    pltpu.sync_copy(ktab_hbm, ktab_v)

    lane = lax.iota(jnp.int32, LANES)

    def _mk_col(h, b):
        for s in range(BLK // LANES):
            p = (lane + s * LANES) * HIST + h
            v = plsc.load_gather(idx_v, [p])
            ocol[b][pl.ds(s * LANES, LANES)] = v
            scol[b][pl.ds(s * LANES, LANES)] = jnp.maximum(v - 2, 0)
        return pltpu.async_copy(pret_hbm.at[scol[b]], rows[b], gsem[b])

    def _fixup(b):
        for j in range(BLK // LANES):
            v = ocol[b][pl.ds(j * LANES, LANES)]

            @pl.when(jnp.min(v) < 2)
            def _():
                m = v < 2
                vs = jnp.minimum(v, 1)
                r16 = lane + j * LANES
                for d in range(DIM):
                    col = jnp.full((LANES,), d, jnp.int32)
                    x = plsc.load_gather(ktab_v, [vs, col])
                    plsc.store_scatter(rows[b], [r16, col], x, mask=m)

    def _transpose(b):
        @plsc.parallel_loop(0, BLK, 1, unroll=4)
        def _(c):
            cc = jnp.full((LANES,), c, jnp.int32)
            for k in range(DIM // LANES):
                v = rows[b][c, pl.ds(k * LANES, LANES)]
                plsc.store_scatter(pblk[b], [lane + k * LANES, cc], v)

    def _write(h, b):
        for q in range(8):
            r0 = ((h * 8 + q) * NW + wid) * 8
            pltpu.async_copy(
                pblk[b].at[pl.ds(q * 8, 8), pl.ds(0, BLK)],
                out_hbm.at[pl.ds(r0, 8)],
                wsem[b],
            )

    def _wait_gather(b):
        pltpu.make_async_copy(pret_hbm.at[scol[b]], rows[b], gsem[b]).wait()

    def _drain_writes(b):
        for q in range(8):
            pltpu.make_async_copy(
                pblk[b].at[pl.ds(q * 8, 8), pl.ds(0, BLK)],
                out_hbm.at[pl.ds(q * 8, 8)],
                wsem[b],
            ).wait()

    _mk_col(0, 0)

    def _iter(i, carry):
        h0 = 2 * i
        _mk_col(h0 + 1, 1)
        _wait_gather(0)
        _fixup(0)
        pl.when(i > 0)(lambda: _drain_writes(0))
        _transpose(0)
        _write(h0, 0)
        def _prefetch():
            _mk_col(h0 + 2, 0)

        pl.when(i + 1 < HIST // 2)(_prefetch)
        _wait_gather(1)
        _fixup(1)
        pl.when(i > 0)(lambda: _drain_writes(1))
        _transpose(1)
        _write(h0 + 1, 1)
        return carry

    lax.fori_loop(0, HIST // 2, _iter, 0)
    _drain_writes(0)
    _drain_writes(1)


def _gather(idx, ktab, pret):
    n = idx.shape[0]
    mesh = plsc.VectorSubcoreMesh(core_axis_name="c", subcore_axis_name="s")
    nrows = n * DIM // 128
    return pl.kernel(
        _body,
        out_type=jax.ShapeDtypeStruct((nrows, 128), jnp.float32),
        mesh=mesh,
        compiler_params=pltpu.CompilerParams(
            needs_layout_passes=False, use_tc_tiling_on_sc=False
        ),
        scratch_types=[
            pltpu.VMEM((BLK * HIST,), jnp.int32),
            pltpu.VMEM((2, DIM), jnp.float32),
            pltpu.VMEM((BLK,), jnp.int32),
            pltpu.VMEM((BLK,), jnp.int32),
            pltpu.VMEM((BLK,), jnp.int32),
            pltpu.VMEM((BLK,), jnp.int32),
            pltpu.VMEM((BLK, DIM), jnp.float32),
            pltpu.VMEM((BLK, DIM), jnp.float32),
            pltpu.VMEM((DIM, BLK + 1), jnp.float32),
            pltpu.VMEM((DIM, BLK + 1), jnp.float32),
            pltpu.SemaphoreType.DMA,
            pltpu.SemaphoreType.DMA,
            pltpu.SemaphoreType.DMA,
            pltpu.SemaphoreType.DMA,
        ],
    )(idx, ktab, pret)


def kernel(inputs, kernel, pretrained):
    b, h = inputs.shape
    idx = inputs.reshape(-1).astype(jnp.int32)
    out2 = _gather(idx, kernel, pretrained)
    o5 = out2.reshape(h, DIM // 8, b // 128, 8, 128)
    return o5.transpose(2, 4, 0, 1, 3).reshape(b, h, DIM)


# speedup vs baseline: 2.7949x; 1.2046x over previous
import functools

import jax
import jax.numpy as jnp
from jax import lax
from jax.experimental import pallas as pl
from jax.experimental.pallas import tpu as pltpu
from jax.experimental.pallas import tpu_sc as plsc

DIM = 64
HIST = 50
NUM_CORES = 2
NUM_SUBCORES = 16
NW = NUM_CORES * NUM_SUBCORES
LANES = 16
BLK = 128


def _body(idx_hbm, ktab_hbm, pret_hbm, out_hbm, idx_v, ktab_v,
          ocol0, ocol1, scol0, scol1,
          rows0, rows1, pblk0, pblk1, gsem0, gsem1, wsem0, wsem1):
    bpw = BLK * HIST
    wid = lax.axis_index("s") * NUM_CORES + lax.axis_index("c")
    base = wid * bpw
    ocol = (ocol0, ocol1)
    scol = (scol0, scol1)
    rows = (rows0, rows1)
    pblk = (pblk0, pblk1)
    gsem = (gsem0, gsem1)
    wsem = (wsem0, wsem1)

    pltpu.sync_copy(idx_hbm.at[pl.ds(base, bpw)], idx_v)
    pltpu.sync_copy(ktab_hbm, ktab_v)

    lane = lax.iota(jnp.int32, LANES)

    def _mk_col(h, b):
        for s in range(BLK // LANES):
            p = (lane + s * LANES) * HIST + h
            v = plsc.load_gather(idx_v, [p])
            ocol[b][pl.ds(s * LANES, LANES)] = v
            scol[b][pl.ds(s * LANES, LANES)] = jnp.maximum(v - 2, 0)
        return pltpu.async_copy(pret_hbm.at[scol[b]], rows[b], gsem[b])

    def _fixup(b):
        mn = ocol[b][pl.ds(0, LANES)]
        for j in range(1, BLK // LANES):
            mn = jnp.minimum(mn, ocol[b][pl.ds(j * LANES, LANES)])

        @pl.when(jnp.min(mn) < 2)
        def _():
            for j in range(BLK // LANES):
                v = ocol[b][pl.ds(j * LANES, LANES)]
                m = v < 2
                vs = jnp.minimum(v, 1)
                r16 = lane + j * LANES
                for d in range(DIM):
                    col = jnp.full((LANES,), d, jnp.int32)
                    x = plsc.load_gather(ktab_v, [vs, col])
                    plsc.store_scatter(rows[b], [r16, col], x, mask=m)

    def _transpose(b):
        @plsc.parallel_loop(0, BLK, 1, unroll=8)
        def _(c):
            cc = jnp.full((LANES,), c, jnp.int32)
            for k in range(DIM // LANES):
                v = rows[b][c, pl.ds(k * LANES, LANES)]
                plsc.store_scatter(pblk[b], [lane + k * LANES, cc], v)

    def _write(h, b):
        for q in range(8):
            r0 = ((h * 8 + q) * NW + wid) * 8
            pltpu.async_copy(
                pblk[b].at[pl.ds(q * 8, 8), pl.ds(0, BLK)],
                out_hbm.at[pl.ds(r0, 8)],
                wsem[b],
            )

    def _wait_gather(b):
        pltpu.make_async_copy(pret_hbm.at[scol[b]], rows[b], gsem[b]).wait()

    def _drain_writes(b):
        for q in range(8):
            pltpu.make_async_copy(
                pblk[b].at[pl.ds(q * 8, 8), pl.ds(0, BLK)],
                out_hbm.at[pl.ds(q * 8, 8)],
                wsem[b],
            ).wait()

    _mk_col(0, 0)

    def _iter(i, carry):
        h0 = 2 * i
        _mk_col(h0 + 1, 1)
        _wait_gather(0)
        _fixup(0)
        pl.when(i > 0)(lambda: _drain_writes(0))
        _transpose(0)
        _write(h0, 0)
        def _prefetch():
            _mk_col(h0 + 2, 0)

        pl.when(i + 1 < HIST // 2)(_prefetch)
        _wait_gather(1)
        _fixup(1)
        pl.when(i > 0)(lambda: _drain_writes(1))
        _transpose(1)
        _write(h0 + 1, 1)
        return carry

    lax.fori_loop(0, HIST // 2, _iter, 0)
    _drain_writes(0)
    _drain_writes(1)


def _gather(idx, ktab, pret):
    n = idx.shape[0]
    mesh = plsc.VectorSubcoreMesh(core_axis_name="c", subcore_axis_name="s")
    nrows = n * DIM // 128
    return pl.kernel(
        _body,
        out_type=jax.ShapeDtypeStruct((nrows, 128), jnp.float32),
        mesh=mesh,
        compiler_params=pltpu.CompilerParams(
            needs_layout_passes=False, use_tc_tiling_on_sc=False
        ),
        scratch_types=[
            pltpu.VMEM((BLK * HIST,), jnp.int32),
            pltpu.VMEM((2, DIM), jnp.float32),
            pltpu.VMEM((BLK,), jnp.int32),
            pltpu.VMEM((BLK,), jnp.int32),
            pltpu.VMEM((BLK,), jnp.int32),
            pltpu.VMEM((BLK,), jnp.int32),
            pltpu.VMEM((BLK, DIM), jnp.float32),
            pltpu.VMEM((BLK, DIM), jnp.float32),
            pltpu.VMEM((DIM, BLK + 1), jnp.float32),
            pltpu.VMEM((DIM, BLK + 1), jnp.float32),
            pltpu.SemaphoreType.DMA,
            pltpu.SemaphoreType.DMA,
            pltpu.SemaphoreType.DMA,
            pltpu.SemaphoreType.DMA,
        ],
    )(idx, ktab, pret)


def kernel(inputs, kernel, pretrained):
    b, h = inputs.shape
    idx = inputs.reshape(-1).astype(jnp.int32)
    out2 = _gather(idx, kernel, pretrained)
    o5 = out2.reshape(h, DIM // 8, b // 128, 8, 128)
    return o5.transpose(2, 4, 0, 1, 3).reshape(b, h, DIM)
